# baseline (speedup 1.0000x reference)
import functools

import jax
import jax.numpy as jnp
from jax import lax
from jax.experimental import pallas as pl
from jax.experimental.pallas import tpu as pltpu
from jax.experimental.pallas import tpu_sc as plsc

NUM_EMB = 1000000
DIM = 64
BATCH = 16384
FIELDS = 26
NW = 32
ROWS_PER_W = BATCH // NW
CHUNK_ROWS = 16
N_CHUNK = ROWS_PER_W // CHUNK_ROWS
NBUF = 2
T_OUTER = N_CHUNK // NBUF


def _make_gather():
    mesh = plsc.VectorSubcoreMesh(core_axis_name="c", subcore_axis_name="s")

    @functools.partial(
        pl.kernel,
        mesh=mesh,
        out_type=jax.ShapeDtypeStruct((BATCH, FIELDS, DIM), jnp.float32),
        scratch_types=[
            pltpu.VMEM((NBUF, CHUNK_ROWS, FIELDS), jnp.int32),
            pltpu.VMEM((NBUF, CHUNK_ROWS, FIELDS, DIM), jnp.float32),
            pltpu.SemaphoreType.DMA,
            pltpu.SemaphoreType.DMA,
            pltpu.SemaphoreType.DMA,
        ],
        compiler_params=pltpu.CompilerParams(use_tc_tiling_on_sc=False),
    )
    def gather_kernel(idx_hbm, table_hbm, out_hbm, idx_v, rows_v,
                      gs0, gs1, osem):
        gsems = [gs0, gs1]
        nc = 2
        wid = lax.axis_index("s") * nc + lax.axis_index("c")
        row_base = wid * ROWS_PER_W

        def load_and_fire(g, b):
            row_off = row_base + g * CHUNK_ROWS
            pltpu.sync_copy(idx_hbm.at[pl.ds(row_off, CHUNK_ROWS)],
                            idx_v.at[b])
            for i in range(CHUNK_ROWS):
                pltpu.async_copy(table_hbm.at[idx_v.at[b, i]],
                                 rows_v.at[b, i], gsems[b])

        def drain_gather(b):
            for i in range(CHUNK_ROWS):
                pltpu.make_async_copy(
                    out_hbm.at[0],
                    rows_v.at[b, i],
                    gsems[b],
                ).wait()

        for b in range(NBUF):
            load_and_fire(b, b)

        def body(t, carry):
            for b in range(NBUF):
                g = t * NBUF + b
                drain_gather(b)
                row_off = row_base + g * CHUNK_ROWS
                cp = pltpu.async_copy(
                    rows_v.at[b],
                    out_hbm.at[pl.ds(row_off, CHUNK_ROWS)],
                    osem,
                )
                next_off = row_base + (g + NBUF) * CHUNK_ROWS
                pltpu.sync_copy(idx_hbm.at[pl.ds(next_off, CHUNK_ROWS)],
                                idx_v.at[b])
                cp.wait()
                for i in range(CHUNK_ROWS):
                    pltpu.async_copy(table_hbm.at[idx_v.at[b, i]],
                                     rows_v.at[b, i], gsems[b])
            return carry

        lax.fori_loop(0, T_OUTER - 1, body, 0)

        stores = []
        for b in range(NBUF):
            g = (T_OUTER - 1) * NBUF + b
            drain_gather(b)
            row_off = row_base + g * CHUNK_ROWS
            stores.append(
                pltpu.async_copy(
                    rows_v.at[b],
                    out_hbm.at[pl.ds(row_off, CHUNK_ROWS)],
                    osem,
                )
            )
        for cp in stores:
            cp.wait()

    return gather_kernel


_gather = _make_gather()


def kernel(tokens_ids, weights):
    return _gather(tokens_ids.astype(jnp.int32), weights)


# speedup vs baseline: 1.0041x; 1.0041x over previous
import functools

import jax
import jax.numpy as jnp
from jax import lax
from jax.experimental import pallas as pl
from jax.experimental.pallas import tpu as pltpu
from jax.experimental.pallas import tpu_sc as plsc

NUM_EMB = 1000000
DIM = 64
BATCH = 16384
FIELDS = 26
ROWS = BATCH * FIELDS
NW = 32
R_PER_W = ROWS // NW
CHUNK = 832
N_CHUNK = R_PER_W // CHUNK
NBUF = 2


def _make_gather():
    mesh = plsc.VectorSubcoreMesh(core_axis_name="c", subcore_axis_name="s")

    @functools.partial(
        pl.kernel,
        mesh=mesh,
        out_type=jax.ShapeDtypeStruct((ROWS, DIM), jnp.float32),
        scratch_types=[
            pltpu.VMEM((R_PER_W,), jnp.int32),
            pltpu.VMEM((NBUF, CHUNK, DIM), jnp.float32),
            pltpu.SemaphoreType.DMA,
            pltpu.SemaphoreType.DMA,
            pltpu.SemaphoreType.DMA,
            pltpu.SemaphoreType.DMA,
        ],
        compiler_params=pltpu.CompilerParams(use_tc_tiling_on_sc=False),
    )
    def gather_kernel(tok_hbm, table_hbm, out_hbm, idx_v, rows_v,
                      gs0, gs1, os0, os1):
        gsems = [gs0, gs1]
        osems = [os0, os1]
        wid = lax.axis_index("s") * 2 + lax.axis_index("c")
        base = wid * R_PER_W

        pltpu.sync_copy(tok_hbm.at[pl.ds(base, R_PER_W)], idx_v)

        def fire_gather(c, b):
            pltpu.async_copy(
                table_hbm.at[idx_v.at[pl.ds(c * CHUNK, CHUNK)]],
                rows_v.at[b],
                gsems[b],
            )

        def drain_gather(b):
            pltpu.make_async_copy(
                table_hbm.at[pl.ds(0, CHUNK)], rows_v.at[b], gsems[b]
            ).wait()

        def fire_store(c, b):
            pltpu.async_copy(
                rows_v.at[b],
                out_hbm.at[pl.ds(base + c * CHUNK, CHUNK)],
                osems[b],
            )

        def drain_store(b):
            pltpu.make_async_copy(
                rows_v.at[b], out_hbm.at[pl.ds(0, CHUNK)], osems[b]
            ).wait()

        fire_gather(0, 0)
        for c in range(N_CHUNK):
            nxt = c + 1
            if nxt < N_CHUNK:
                b_nxt = nxt % NBUF
                if nxt >= NBUF:
                    drain_store(b_nxt)
                fire_gather(nxt, b_nxt)
            b = c % NBUF
            drain_gather(b)
            fire_store(c, b)
        for c in range(N_CHUNK - NBUF, N_CHUNK):
            drain_store(c % NBUF)

    return gather_kernel


_gather = _make_gather()


def kernel(tokens_ids, weights):
    tok_flat = tokens_ids.astype(jnp.int32).reshape(ROWS)
    out = _gather(tok_flat, weights)
    return out.reshape(BATCH, FIELDS, DIM)
